# baseline (speedup 1.0000x reference)
import jax
import jax.numpy as jnp
from jax import lax
from jax.experimental import pallas as pl
from jax.experimental.pallas import tpu as pltpu
from jax.experimental.pallas import tpu_sc as plsc

_ITEM_NUM = 1000000
_USER_NUM = 100000
_IH = 48
_UH = 16
_OH = _IH + _UH
_SSE_PROB = 0.08
_B = 4096
_L = 200

_NC = 2
_NS = 16
_NW = _NC * _NS
_B_PER_W = _B // _NW
_CB = 1
_R = _CB * _L
_NCHUNK = _B_PER_W // _CB
_ROWS_PER_W = _B_PER_W * _L
_NBUF = 3


def _sse_uids(user_ids):
    key = jax.random.key(42)
    ku, kr = jax.random.split(key)
    probs = jax.random.uniform(ku, user_ids.shape)
    rand_ids = jax.random.randint(kr, user_ids.shape, 1, _USER_NUM + 1)
    rand_ids = rand_ids.astype(user_ids.dtype)
    return jnp.where(probs < _SSE_PROB, rand_ids, user_ids)


def _body(seqs_hbm, uidx_hbm, item_hbm, user_hbm, out_hbm,
          idxi_v, uidx_v, ulocal_v, itm_v, rows_v, gsems, wsems):
    wid = lax.axis_index("s") * _NC + lax.axis_index("c")
    base_row = wid * _ROWS_PER_W

    pltpu.sync_copy(seqs_hbm.at[pl.ds(base_row, _ROWS_PER_W)], idxi_v)
    pltpu.sync_copy(uidx_hbm.at[pl.ds(wid * _B_PER_W, _B_PER_W)], uidx_v)
    pltpu.async_copy(user_hbm.at[uidx_v], ulocal_v, gsems.at[0, 0]).wait()

    _SPLITS = (0, 56, 112, 168, _R)
    _NSTR = len(_SPLITS) - 1

    def fire(ci, k):
        off = ci * _R
        for s in range(_NSTR):
            o, n = _SPLITS[s], _SPLITS[s + 1] - _SPLITS[s]
            pltpu.async_copy(
                item_hbm.at[idxi_v.at[pl.ds(off + o, n)]],
                itm_v.at[k, pl.ds(o, n)], gsems.at[k, s])

    def wait_gather(ci, k):
        off = ci * _R
        for s in range(_NSTR):
            o, n = _SPLITS[s], _SPLITS[s + 1] - _SPLITS[s]
            pltpu.make_async_copy(
                item_hbm.at[idxi_v.at[pl.ds(off + o, n)]],
                itm_v.at[k, pl.ds(o, n)], gsems.at[k, s]).wait()

    def assemble(ci, k):
        for half in range(_CB):
            uvec = ulocal_v[ci * _CB + half, :]

            def row_body(r, carry):
                rr = half * _L + r
                rows_v[k, rr, 0:16] = itm_v[k, rr, 0:16]
                rows_v[k, rr, 16:32] = itm_v[k, rr, 16:32]
                rows_v[k, rr, 32:48] = itm_v[k, rr, 32:48]
                rows_v[k, rr, 48:64] = uvec
                return carry

            lax.fori_loop(0, _L, row_body, 0)

    def fire_write(ci, k):
        row0 = base_row + ci * _R
        pltpu.async_copy(rows_v.at[k], out_hbm.at[pl.ds(row0, _R)],
                         wsems.at[k])

    def wait_write(ci, k):
        row0 = base_row + ci * _R
        pltpu.make_async_copy(rows_v.at[k], out_hbm.at[pl.ds(row0, _R)],
                              wsems.at[k]).wait()

    for k in range(_NBUF):
        fire(k, k)

    def group_body(g, carry):
        for k in range(_NBUF):
            ci = g * _NBUF + k

            wait_gather(ci, k)

            @pl.when(ci >= _NBUF)
            def _():
                wait_write(ci - _NBUF, k)

            assemble(ci, k)

            @pl.when(ci + _NBUF < _NCHUNK)
            def _():
                fire(ci + _NBUF, k)

            fire_write(ci, k)
        return carry

    lax.fori_loop(0, _NCHUNK // _NBUF, group_body, 0)

    for ci in range((_NCHUNK // _NBUF) * _NBUF, _NCHUNK):
        k = ci % _NBUF
        wait_gather(ci, k)
        wait_write(ci - _NBUF, k)
        assemble(ci, k)
        fire_write(ci, k)

    for ci in range(_NCHUNK - _NBUF, _NCHUNK):
        wait_write(ci, ci % _NBUF)


@jax.jit
def _sc_embed(seqs1d, uidx, item_table, user_table):
    mesh = plsc.VectorSubcoreMesh(core_axis_name="c", subcore_axis_name="s")
    f = pl.kernel(
        _body,
        out_type=jax.ShapeDtypeStruct((_B * _L, _OH), jnp.float32),
        mesh=mesh,
        scratch_types=[
            pltpu.VMEM((_ROWS_PER_W,), jnp.int32),
            pltpu.VMEM((_B_PER_W,), jnp.int32),
            pltpu.VMEM((_B_PER_W, _UH), jnp.float32),
            pltpu.VMEM((_NBUF, _R, _IH), jnp.float32),
            pltpu.VMEM((_NBUF, _R, _OH), jnp.float32),
            pltpu.SemaphoreType.DMA((_NBUF, 4)),
            pltpu.SemaphoreType.DMA((_NBUF,)),
        ],
        compiler_params=pltpu.CompilerParams(use_tc_tiling_on_sc=False),
    )
    return f(seqs1d, uidx, item_table, user_table)


def kernel(log_seqs, user_ids, item_table, user_table):
    uids = _sse_uids(user_ids).astype(jnp.int32)
    seqs1d = log_seqs.reshape(-1).astype(jnp.int32)
    item2d = jax.lax.optimization_barrier(jnp.swapaxes(item_table, 0, 1))
    item2d = jnp.swapaxes(item2d, 0, 1)
    user2d = jax.lax.optimization_barrier(jnp.swapaxes(user_table, 0, 1))
    user2d = jnp.swapaxes(user2d, 0, 1)
    out2d = _sc_embed(seqs1d, uids, item2d, user2d)
    return out2d.reshape(_B, _L, _OH)


# speedup vs baseline: 1.0521x; 1.0521x over previous
import jax
import jax.numpy as jnp
from jax import lax
from jax.experimental import pallas as pl
from jax.experimental.pallas import tpu as pltpu
from jax.experimental.pallas import tpu_sc as plsc

_ITEM_NUM = 1000000
_USER_NUM = 100000
_IH = 48
_UH = 16
_OH = _IH + _UH
_SSE_PROB = 0.08
_B = 4096
_L = 200

_NC = 2
_NS = 16
_NW = _NC * _NS
_B_PER_W = _B // _NW
_CB = 1
_R = _CB * _L
_NCHUNK = _B_PER_W // _CB
_ROWS_PER_W = _B_PER_W * _L
_NBUF = 4


def _sse_uids(user_ids):
    key = jax.random.key(42)
    ku, kr = jax.random.split(key)
    probs = jax.random.uniform(ku, user_ids.shape)
    rand_ids = jax.random.randint(kr, user_ids.shape, 1, _USER_NUM + 1)
    rand_ids = rand_ids.astype(user_ids.dtype)
    return jnp.where(probs < _SSE_PROB, rand_ids, user_ids)


def _body(seqs_hbm, uidx_hbm, item_hbm, user_hbm, out_hbm,
          idxi_v, uidx_v, ulocal_v, itm_v, ubuf_v, gsems, wsems):
    wid = lax.axis_index("s") * _NC + lax.axis_index("c")
    base_row = wid * _ROWS_PER_W

    pltpu.sync_copy(seqs_hbm.at[pl.ds(base_row, _ROWS_PER_W)], idxi_v)
    pltpu.sync_copy(uidx_hbm.at[pl.ds(wid * _B_PER_W, _B_PER_W)], uidx_v)
    pltpu.async_copy(user_hbm.at[uidx_v], ulocal_v, gsems.at[0, 0]).wait()

    _SPLITS = (0, 56, 112, 168, _R)
    _NSTR = len(_SPLITS) - 1

    def fire(ci, k):
        off = ci * _R
        for s in range(_NSTR):
            o, n = _SPLITS[s], _SPLITS[s + 1] - _SPLITS[s]
            pltpu.async_copy(
                item_hbm.at[idxi_v.at[pl.ds(off + o, n)]],
                itm_v.at[k, pl.ds(o, n)], gsems.at[k, s])

    def wait_gather(ci, k):
        off = ci * _R
        for s in range(_NSTR):
            o, n = _SPLITS[s], _SPLITS[s + 1] - _SPLITS[s]
            pltpu.make_async_copy(
                item_hbm.at[idxi_v.at[pl.ds(off + o, n)]],
                itm_v.at[k, pl.ds(o, n)], gsems.at[k, s]).wait()

    def assemble(ci, k):
        uvec = ulocal_v[ci, :]

        def row_body(r, carry):
            ubuf_v[k, r, :] = uvec
            return carry

        lax.fori_loop(0, _R, row_body, 0)

    def fire_write(ci, k):
        row0 = base_row + ci * _R
        pltpu.async_copy(itm_v.at[k],
                         out_hbm.at[pl.ds(row0, _R), pl.ds(0, _IH)],
                         wsems.at[k, 0])
        pltpu.async_copy(ubuf_v.at[k],
                         out_hbm.at[pl.ds(row0, _R), pl.ds(_IH, _UH)],
                         wsems.at[k, 1])

    def wait_write(ci, k):
        row0 = base_row + ci * _R
        pltpu.make_async_copy(itm_v.at[k],
                              out_hbm.at[pl.ds(row0, _R), pl.ds(0, _IH)],
                              wsems.at[k, 0]).wait()
        pltpu.make_async_copy(ubuf_v.at[k],
                              out_hbm.at[pl.ds(row0, _R), pl.ds(_IH, _UH)],
                              wsems.at[k, 1]).wait()

    for k in range(_NBUF):
        fire(k, k)

    def group_body(g, carry):
        for k in range(_NBUF):
            ci = g * _NBUF + k

            wait_gather(ci, k)
            assemble(ci, k)
            fire_write(ci, k)

            wait_write(ci, k)

            @pl.when(ci + _NBUF < _NCHUNK)
            def _():
                fire(ci + _NBUF, k)
        return carry

    lax.fori_loop(0, _NCHUNK // _NBUF, group_body, 0)

    for ci in range((_NCHUNK // _NBUF) * _NBUF, _NCHUNK):
        k = ci % _NBUF
        wait_gather(ci, k)
        assemble(ci, k)
        fire_write(ci, k)
        wait_write(ci, k)


@jax.jit
def _sc_embed(seqs1d, uidx, item_table, user_table):
    mesh = plsc.VectorSubcoreMesh(core_axis_name="c", subcore_axis_name="s")
    f = pl.kernel(
        _body,
        out_type=jax.ShapeDtypeStruct((_B * _L, _OH), jnp.float32),
        mesh=mesh,
        scratch_types=[
            pltpu.VMEM((_ROWS_PER_W,), jnp.int32),
            pltpu.VMEM((_B_PER_W,), jnp.int32),
            pltpu.VMEM((_B_PER_W, _UH), jnp.float32),
            pltpu.VMEM((_NBUF, _R, _IH), jnp.float32),
            pltpu.VMEM((_NBUF, _R, _UH), jnp.float32),
            pltpu.SemaphoreType.DMA((_NBUF, 4)),
            pltpu.SemaphoreType.DMA((_NBUF, 2)),
        ],
        compiler_params=pltpu.CompilerParams(use_tc_tiling_on_sc=False),
    )
    return f(seqs1d, uidx, item_table, user_table)


def kernel(log_seqs, user_ids, item_table, user_table):
    uids = _sse_uids(user_ids).astype(jnp.int32)
    seqs1d = log_seqs.reshape(-1).astype(jnp.int32)
    item2d = jax.lax.optimization_barrier(jnp.swapaxes(item_table, 0, 1))
    item2d = jnp.swapaxes(item2d, 0, 1)
    user2d = jax.lax.optimization_barrier(jnp.swapaxes(user_table, 0, 1))
    user2d = jnp.swapaxes(user2d, 0, 1)
    out2d = _sc_embed(seqs1d, uids, item2d, user2d)
    return out2d.reshape(_B, _L, _OH)


# speedup vs baseline: 1.0528x; 1.0007x over previous
import jax
import jax.numpy as jnp
from jax import lax
from jax.experimental import pallas as pl
from jax.experimental.pallas import tpu as pltpu
from jax.experimental.pallas import tpu_sc as plsc

_ITEM_NUM = 1000000
_USER_NUM = 100000
_IH = 48
_UH = 16
_OH = _IH + _UH
_SSE_PROB = 0.08
_B = 4096
_L = 200

_NC = 2
_NS = 16
_NW = _NC * _NS
_B_PER_W = _B // _NW
_CB = 2
_R = _CB * _L
_NCHUNK = _B_PER_W // _CB
_ROWS_PER_W = _B_PER_W * _L
_NBUF = 3


def _sse_uids(user_ids):
    key = jax.random.key(42)
    ku, kr = jax.random.split(key)
    probs = jax.random.uniform(ku, user_ids.shape)
    rand_ids = jax.random.randint(kr, user_ids.shape, 1, _USER_NUM + 1)
    rand_ids = rand_ids.astype(user_ids.dtype)
    return jnp.where(probs < _SSE_PROB, rand_ids, user_ids)


def _body(seqs_hbm, uidx_hbm, item_hbm, user_hbm, out_hbm,
          idxi_v, uidx_v, ulocal_v, itm_v, ubuf_v, gsems, wsems):
    wid = lax.axis_index("s") * _NC + lax.axis_index("c")
    base_row = wid * _ROWS_PER_W

    pltpu.sync_copy(seqs_hbm.at[pl.ds(base_row, _ROWS_PER_W)], idxi_v)
    pltpu.sync_copy(uidx_hbm.at[pl.ds(wid * _B_PER_W, _B_PER_W)], uidx_v)
    pltpu.async_copy(user_hbm.at[uidx_v], ulocal_v, gsems.at[0, 0]).wait()

    _SPLITS = (0, 104, 208, 304, _R)
    _NSTR = len(_SPLITS) - 1

    def fire(ci, k):
        off = ci * _R
        for s in range(_NSTR):
            o, n = _SPLITS[s], _SPLITS[s + 1] - _SPLITS[s]
            pltpu.async_copy(
                item_hbm.at[idxi_v.at[pl.ds(off + o, n)]],
                itm_v.at[k, pl.ds(o, n)], gsems.at[k, s])

    def wait_gather(ci, k):
        off = ci * _R
        for s in range(_NSTR):
            o, n = _SPLITS[s], _SPLITS[s + 1] - _SPLITS[s]
            pltpu.make_async_copy(
                item_hbm.at[idxi_v.at[pl.ds(off + o, n)]],
                itm_v.at[k, pl.ds(o, n)], gsems.at[k, s]).wait()

    def assemble(ci, k):
        for half in range(_CB):
            uvec = ulocal_v[ci * _CB + half, :]

            def row_body(r, carry):
                ubuf_v[k, half * _L + r, :] = uvec
                return carry

            lax.fori_loop(0, _L, row_body, 0)

    def fire_write(ci, k):
        row0 = base_row + ci * _R
        pltpu.async_copy(itm_v.at[k],
                         out_hbm.at[pl.ds(row0, _R), pl.ds(0, _IH)],
                         wsems.at[k, 0])
        pltpu.async_copy(ubuf_v.at[k],
                         out_hbm.at[pl.ds(row0, _R), pl.ds(_IH, _UH)],
                         wsems.at[k, 1])

    def wait_write(ci, k):
        row0 = base_row + ci * _R
        pltpu.make_async_copy(itm_v.at[k],
                              out_hbm.at[pl.ds(row0, _R), pl.ds(0, _IH)],
                              wsems.at[k, 0]).wait()
        pltpu.make_async_copy(ubuf_v.at[k],
                              out_hbm.at[pl.ds(row0, _R), pl.ds(_IH, _UH)],
                              wsems.at[k, 1]).wait()

    for k in range(_NBUF):
        fire(k, k)

    def group_body(g, carry):
        for k in range(_NBUF):
            ci = g * _NBUF + k

            wait_gather(ci, k)
            assemble(ci, k)
            fire_write(ci, k)

            wait_write(ci, k)

            @pl.when(ci + _NBUF < _NCHUNK)
            def _():
                fire(ci + _NBUF, k)
        return carry

    lax.fori_loop(0, _NCHUNK // _NBUF, group_body, 0)

    for ci in range((_NCHUNK // _NBUF) * _NBUF, _NCHUNK):
        k = ci % _NBUF
        wait_gather(ci, k)
        assemble(ci, k)
        fire_write(ci, k)
        wait_write(ci, k)


@jax.jit
def _sc_embed(seqs1d, uidx, item_table, user_table):
    mesh = plsc.VectorSubcoreMesh(core_axis_name="c", subcore_axis_name="s")
    f = pl.kernel(
        _body,
        out_type=jax.ShapeDtypeStruct((_B * _L, _OH), jnp.float32),
        mesh=mesh,
        scratch_types=[
            pltpu.VMEM((_ROWS_PER_W,), jnp.int32),
            pltpu.VMEM((_B_PER_W,), jnp.int32),
            pltpu.VMEM((_B_PER_W, _UH), jnp.float32),
            pltpu.VMEM((_NBUF, _R, _IH), jnp.float32),
            pltpu.VMEM((_NBUF, _R, _UH), jnp.float32),
            pltpu.SemaphoreType.DMA((_NBUF, 4)),
            pltpu.SemaphoreType.DMA((_NBUF, 2)),
        ],
        compiler_params=pltpu.CompilerParams(use_tc_tiling_on_sc=False),
    )
    return f(seqs1d, uidx, item_table, user_table)


def kernel(log_seqs, user_ids, item_table, user_table):
    uids = _sse_uids(user_ids).astype(jnp.int32)
    seqs1d = log_seqs.reshape(-1).astype(jnp.int32)
    item2d = jax.lax.optimization_barrier(jnp.swapaxes(item_table, 0, 1))
    item2d = jnp.swapaxes(item2d, 0, 1)
    user2d = jax.lax.optimization_barrier(jnp.swapaxes(user_table, 0, 1))
    user2d = jnp.swapaxes(user2d, 0, 1)
    out2d = _sc_embed(seqs1d, uids, item2d, user2d)
    return out2d.reshape(_B, _L, _OH)
